# baseline (speedup 1.0000x reference)
import functools

import jax
import jax.numpy as jnp
from jax import lax
from jax.experimental import pallas as pl
from jax.experimental.pallas import tpu as pltpu
from jax.experimental.pallas import tpu_sc as plsc

NUM_K = 8
NUM_LABELS = 100
QB = 1024
KB = 2048
KPAD = 100352
NSTEPS = KPAD // KB

_NEG_INF = float("-inf")
_IMAX = 2**31 - 1


def _topk_body(x2_ref, qnorm_ref, tT_ref, knorm_ref, vals_ref, idxs_ref,
               c_ref):
    t = pl.program_id(0)

    @pl.when(t == 0)
    def _init():
        vals_ref[...] = jnp.full((QB, NUM_K), _NEG_INF, jnp.float32)
        idxs_ref[...] = jnp.full((QB, NUM_K), _IMAX, jnp.int32)

    cross2 = jnp.dot(x2_ref[...], tT_ref[...],
                     preferred_element_type=jnp.float32)
    s = cross2 - jnp.sqrt(qnorm_ref[...] + knorm_ref[0])
    c_ref[...] = s
    col = lax.broadcasted_iota(jnp.int32, (QB, KB), 1)

    m0 = jnp.max(s, axis=1, keepdims=True)
    go0 = jnp.any(m0 > vals_ref[:, NUM_K - 1:])

    def _cond(carry):
        go, _ = carry
        return go

    def _body(carry):
        _, m = carry
        c = c_ref[...]
        il = jnp.argmax(c, axis=1).astype(jnp.int32)[:, None]
        cn = jnp.where(col == il, _NEG_INF, c)
        m2 = jnp.max(cn, axis=1, keepdims=True)
        im = il + t * KB
        v8 = vals_ref[...]
        i8 = idxs_ref[...]
        ge = (v8 > m) | ((v8 == m) & (i8 < im))
        r = jnp.sum(ge.astype(jnp.int32), axis=1, keepdims=True)
        pos = lax.broadcasted_iota(jnp.int32, (QB, NUM_K), 1)
        vsh = jnp.concatenate([m, v8[:, :NUM_K - 1]], axis=1)
        ish = jnp.concatenate([im, i8[:, :NUM_K - 1]], axis=1)
        vals_ref[...] = jnp.where(pos < r, v8, jnp.where(pos == r, m, vsh))
        idxs_ref[...] = jnp.where(pos < r, i8, jnp.where(pos == r, im, ish))
        c_ref[...] = cn
        go2 = jnp.any(m2 > vals_ref[:, NUM_K - 1:])
        return go2, m2

    lax.while_loop(_cond, _body, (go0, m0))


def _topk(x, qnorm, tT, knorm3):
    return pl.pallas_call(
        _topk_body,
        grid=(NSTEPS,),
        in_specs=[
            pl.BlockSpec((QB, 16), lambda t: (0, 0)),
            pl.BlockSpec((QB, 1), lambda t: (0, 0)),
            pl.BlockSpec((16, KB), lambda t: (0, t)),
            pl.BlockSpec((1, 1, KB), lambda t: (t, 0, 0)),
        ],
        out_specs=[
            pl.BlockSpec((QB, NUM_K), lambda t: (0, 0)),
            pl.BlockSpec((QB, NUM_K), lambda t: (0, 0)),
        ],
        out_shape=[
            jax.ShapeDtypeStruct((QB, NUM_K), jnp.float32),
            jax.ShapeDtypeStruct((QB, NUM_K), jnp.int32),
        ],
        scratch_shapes=[pltpu.VMEM((QB, KB), jnp.float32)],
        compiler_params=pltpu.CompilerParams(
            dimension_semantics=("arbitrary",),
        ),
    )(x, qnorm, tT, knorm3)


_NC, _NS, _L = 2, 16, 16
_NW = _NC * _NS
_CPW = QB // _NW
_LPAD = 100352


def _label_onehot_body(labels_hbm, idx_hbm, zeros_hbm, out_hbm,
                       shared_v, labels_v, idx_v, out_v):
    sid = lax.axis_index("s")
    wid = sid * _NC + lax.axis_index("c")
    base = wid * _CPW

    @pl.when(sid == 0)
    def _stage():
        pltpu.sync_copy(labels_hbm, shared_v)

    plsc.subcore_barrier()
    pltpu.sync_copy(shared_v, labels_v)
    for m in range(NUM_K):
        pltpu.sync_copy(idx_hbm.at[pl.ds(m * QB + base, _CPW)], idx_v.at[m])
    pltpu.sync_copy(zeros_hbm.at[pl.ds(base, _CPW)], out_v)
    lane = lax.iota(jnp.int32, _L)
    ones = jnp.full((_L,), 1.0, jnp.float32)
    for chunk in range(_CPW // _L):
        acc = jnp.zeros((_L,), jnp.int32)
        for m in range(NUM_K):
            iv = idx_v[m, pl.ds(chunk * _L, _L)]
            acc = acc + plsc.load_gather(labels_v, [iv])
        val = lax.shift_right_arithmetic(acc, 3)
        rows = chunk * _L + lane
        plsc.store_scatter(out_v, [rows, val], ones)
    pltpu.sync_copy(out_v, out_hbm.at[pl.ds(base, _CPW)])


def _label_onehot(labels, idx_flat, zeros):
    return pl.kernel(
        _label_onehot_body,
        out_type=jax.ShapeDtypeStruct((QB, NUM_LABELS), jnp.float32),
        mesh=plsc.VectorSubcoreMesh(core_axis_name="c", subcore_axis_name="s"),
        compiler_params=pltpu.CompilerParams(needs_layout_passes=False),
        scratch_types=[
            pltpu.VMEM_SHARED((_LPAD,), jnp.int32),
            pltpu.VMEM((_LPAD,), jnp.int32),
            pltpu.VMEM((NUM_K, _CPW), jnp.int32),
            pltpu.VMEM((_CPW, NUM_LABELS), jnp.float32),
        ],
    )(labels, idx_flat, zeros)


def kernel(x_input, train_inputs, train_labels, features):
    weighted = jnp.multiply(features, train_inputs)
    qnorm = jnp.sum(jnp.square(x_input), axis=1)[:, None]
    knorm = jnp.sum(jnp.square(weighted), axis=1)
    knorm_p = jnp.pad(knorm, (0, KPAD - knorm.shape[0]),
                      constant_values=jnp.inf)
    knorm3 = knorm_p.reshape(NSTEPS, 1, KB)
    tT = jnp.pad(train_inputs, ((0, KPAD - train_inputs.shape[0]), (0, 0))).T

    vals, idxs = _topk(x_input + x_input, qnorm, tT, knorm3)

    labels_p = jnp.pad(train_labels, (0, _LPAD - train_labels.shape[0]))
    one_hot = _label_onehot(labels_p, idxs.reshape(-1),
                            jnp.zeros((QB, NUM_LABELS), jnp.float32))
    return one_hot, vals, idxs


# speedup vs baseline: 1.1256x; 1.1256x over previous
import functools

import jax
import jax.numpy as jnp
from jax import lax
from jax.experimental import pallas as pl
from jax.experimental.pallas import tpu as pltpu
from jax.experimental.pallas import tpu_sc as plsc

NUM_K = 8
NUM_LABELS = 100
QB = 1024
KB = 1792
KPAD = 100352
NSTEPS = KPAD // KB

_NEG_INF = float("-inf")
_IMAX = 2**31 - 1


def _topk_body(x2_ref, qnorm_ref, tT_ref, knorm_ref, vals_ref, idxs_ref,
               c_ref):
    t = pl.program_id(0)

    @pl.when(t == 0)
    def _init():
        vals_ref[...] = jnp.full((QB, NUM_K), _NEG_INF, jnp.float32)
        idxs_ref[...] = jnp.full((QB, NUM_K), _IMAX, jnp.int32)

    cross2 = jnp.dot(x2_ref[...], tT_ref[...],
                     preferred_element_type=jnp.float32)
    s = cross2 - jnp.sqrt(qnorm_ref[...] + knorm_ref[0])
    c_ref[...] = s
    col = lax.broadcasted_iota(jnp.int32, (QB, KB), 1)

    m0 = jnp.max(s, axis=1, keepdims=True)
    go0 = jnp.any(m0 > vals_ref[:, NUM_K - 1:])

    def _cond(carry):
        go, _ = carry
        return go

    def _body(carry):
        _, m = carry
        c = c_ref[...]
        il = jnp.min(jnp.where(c == m, col, _IMAX), axis=1, keepdims=True)
        cn = jnp.where(col == il, _NEG_INF, c)
        m2 = jnp.max(cn, axis=1, keepdims=True)
        im = il + t * KB
        v8 = vals_ref[...]
        i8 = idxs_ref[...]
        ge = (v8 > m) | ((v8 == m) & (i8 < im))
        r = jnp.sum(ge.astype(jnp.int32), axis=1, keepdims=True)
        pos = lax.broadcasted_iota(jnp.int32, (QB, NUM_K), 1)
        vsh = jnp.concatenate([m, v8[:, :NUM_K - 1]], axis=1)
        ish = jnp.concatenate([im, i8[:, :NUM_K - 1]], axis=1)
        vals_ref[...] = jnp.where(pos < r, v8, jnp.where(pos == r, m, vsh))
        idxs_ref[...] = jnp.where(pos < r, i8, jnp.where(pos == r, im, ish))
        c_ref[...] = cn
        go2 = jnp.any(m2 > vals_ref[:, NUM_K - 1:])
        return go2, m2

    lax.while_loop(_cond, _body, (go0, m0))


def _topk(x, qnorm, tT, knorm3):
    return pl.pallas_call(
        _topk_body,
        grid=(NSTEPS,),
        in_specs=[
            pl.BlockSpec((QB, 16), lambda t: (0, 0)),
            pl.BlockSpec((QB, 1), lambda t: (0, 0)),
            pl.BlockSpec((16, KB), lambda t: (0, t)),
            pl.BlockSpec((1, 1, KB), lambda t: (t, 0, 0)),
        ],
        out_specs=[
            pl.BlockSpec((QB, NUM_K), lambda t: (0, 0)),
            pl.BlockSpec((QB, NUM_K), lambda t: (0, 0)),
        ],
        out_shape=[
            jax.ShapeDtypeStruct((QB, NUM_K), jnp.float32),
            jax.ShapeDtypeStruct((QB, NUM_K), jnp.int32),
        ],
        scratch_shapes=[pltpu.VMEM((QB, KB), jnp.float32)],
        compiler_params=pltpu.CompilerParams(
            dimension_semantics=("arbitrary",),
        ),
    )(x, qnorm, tT, knorm3)


_NC, _NS, _L = 2, 16, 16
_NW = _NC * _NS
_CPW = QB // _NW
_LPAD = 100352


def _label_onehot_body(labels_hbm, idx_hbm, zeros_hbm, out_hbm,
                       shared_v, labels_v, idx_v, out_v):
    sid = lax.axis_index("s")
    wid = sid * _NC + lax.axis_index("c")
    base = wid * _CPW

    @pl.when(sid == 0)
    def _stage():
        pltpu.sync_copy(labels_hbm, shared_v)

    plsc.subcore_barrier()
    pltpu.sync_copy(shared_v, labels_v)
    for m in range(NUM_K):
        pltpu.sync_copy(idx_hbm.at[pl.ds(m * QB + base, _CPW)], idx_v.at[m])
    pltpu.sync_copy(zeros_hbm.at[pl.ds(base, _CPW)], out_v)
    lane = lax.iota(jnp.int32, _L)
    ones = jnp.full((_L,), 1.0, jnp.float32)
    for chunk in range(_CPW // _L):
        acc = jnp.zeros((_L,), jnp.int32)
        for m in range(NUM_K):
            iv = idx_v[m, pl.ds(chunk * _L, _L)]
            acc = acc + plsc.load_gather(labels_v, [iv])
        val = lax.shift_right_arithmetic(acc, 3)
        rows = chunk * _L + lane
        plsc.store_scatter(out_v, [rows, val], ones)
    pltpu.sync_copy(out_v, out_hbm.at[pl.ds(base, _CPW)])


def _label_onehot(labels, idx_flat, zeros):
    return pl.kernel(
        _label_onehot_body,
        out_type=jax.ShapeDtypeStruct((QB, NUM_LABELS), jnp.float32),
        mesh=plsc.VectorSubcoreMesh(core_axis_name="c", subcore_axis_name="s"),
        compiler_params=pltpu.CompilerParams(needs_layout_passes=False),
        scratch_types=[
            pltpu.VMEM_SHARED((_LPAD,), jnp.int32),
            pltpu.VMEM((_LPAD,), jnp.int32),
            pltpu.VMEM((NUM_K, _CPW), jnp.int32),
            pltpu.VMEM((_CPW, NUM_LABELS), jnp.float32),
        ],
    )(labels, idx_flat, zeros)


def kernel(x_input, train_inputs, train_labels, features):
    weighted = jnp.multiply(features, train_inputs)
    qnorm = jnp.sum(jnp.square(x_input), axis=1)[:, None]
    knorm = jnp.sum(jnp.square(weighted), axis=1)
    knorm_p = jnp.pad(knorm, (0, KPAD - knorm.shape[0]),
                      constant_values=jnp.inf)
    knorm3 = knorm_p.reshape(NSTEPS, 1, KB)
    tT = jnp.pad(train_inputs, ((0, KPAD - train_inputs.shape[0]), (0, 0))).T

    vals, idxs = _topk(x_input + x_input, qnorm, tT, knorm3)

    labels_p = jnp.pad(train_labels, (0, _LPAD - train_labels.shape[0]))
    one_hot = _label_onehot(labels_p, idxs.reshape(-1),
                            jnp.zeros((QB, NUM_LABELS), jnp.float32))
    return one_hot, vals, idxs


# speedup vs baseline: 1.1416x; 1.0142x over previous
import functools

import jax
import jax.numpy as jnp
from jax import lax
from jax.experimental import pallas as pl
from jax.experimental.pallas import tpu as pltpu
from jax.experimental.pallas import tpu_sc as plsc

NUM_K = 8
NUM_LABELS = 100
QB = 1024
KB = 2048
KPAD = 100352
NSTEPS = KPAD // KB

_NEG_INF = float("-inf")
_IMAX = 2**31 - 1


def _topk_body(x2_ref, qnorm_ref, tT_ref, knorm_ref, vals_ref, idxs_ref,
               c_ref):
    t = pl.program_id(0)

    @pl.when(t == 0)
    def _init():
        vals_ref[...] = jnp.full((QB, NUM_K), _NEG_INF, jnp.float32)
        idxs_ref[...] = jnp.full((QB, NUM_K), _IMAX, jnp.int32)

    cross2 = jnp.dot(x2_ref[...], tT_ref[...],
                     preferred_element_type=jnp.float32)
    s = cross2 - jnp.sqrt(qnorm_ref[...] + knorm_ref[0])
    c_ref[...] = s
    col = lax.broadcasted_iota(jnp.int32, (QB, KB), 1)

    m0 = jnp.max(s, axis=1, keepdims=True)
    go0 = jnp.any(m0 > vals_ref[:, NUM_K - 1:])

    def _cond(carry):
        go, _ = carry
        return go

    def _body(carry):
        _, m = carry
        c = c_ref[...]
        il = jnp.min(jnp.where(c == m, col, _IMAX), axis=1, keepdims=True)
        cn = jnp.where(col == il, _NEG_INF, c)
        m2 = jnp.max(cn, axis=1, keepdims=True)
        im = il + t * KB
        v8 = vals_ref[...]
        i8 = idxs_ref[...]
        ge = (v8 > m) | ((v8 == m) & (i8 < im))
        r = jnp.sum(ge.astype(jnp.int32), axis=1, keepdims=True)
        pos = lax.broadcasted_iota(jnp.int32, (QB, NUM_K), 1)
        vsh = jnp.concatenate([m, v8[:, :NUM_K - 1]], axis=1)
        ish = jnp.concatenate([im, i8[:, :NUM_K - 1]], axis=1)
        vals_ref[...] = jnp.where(pos < r, v8, jnp.where(pos == r, m, vsh))
        idxs_ref[...] = jnp.where(pos < r, i8, jnp.where(pos == r, im, ish))
        c_ref[...] = cn
        go2 = jnp.any(m2 > vals_ref[:, NUM_K - 1:])
        return go2, m2

    lax.while_loop(_cond, _body, (go0, m0))


def _topk(x, qnorm, tT, knorm3):
    return pl.pallas_call(
        _topk_body,
        grid=(NSTEPS,),
        in_specs=[
            pl.BlockSpec((QB, 16), lambda t: (0, 0)),
            pl.BlockSpec((QB, 1), lambda t: (0, 0)),
            pl.BlockSpec((16, KB), lambda t: (0, t)),
            pl.BlockSpec((1, 1, KB), lambda t: (t, 0, 0)),
        ],
        out_specs=[
            pl.BlockSpec((QB, NUM_K), lambda t: (0, 0)),
            pl.BlockSpec((QB, NUM_K), lambda t: (0, 0)),
        ],
        out_shape=[
            jax.ShapeDtypeStruct((QB, NUM_K), jnp.float32),
            jax.ShapeDtypeStruct((QB, NUM_K), jnp.int32),
        ],
        scratch_shapes=[pltpu.VMEM((QB, KB), jnp.float32)],
        compiler_params=pltpu.CompilerParams(
            dimension_semantics=("arbitrary",),
        ),
    )(x, qnorm, tT, knorm3)


_NC, _NS, _L = 2, 16, 16
_NW = _NC * _NS
_CPW = QB // _NW
_LPAD = 100352


def _label_onehot_body(labels_hbm, idx_hbm, zeros_hbm, out_hbm,
                       shared_v, labels_v, idx_v, out_v):
    sid = lax.axis_index("s")
    wid = sid * _NC + lax.axis_index("c")
    base = wid * _CPW

    @pl.when(sid == 0)
    def _stage():
        pltpu.sync_copy(labels_hbm, shared_v)

    plsc.subcore_barrier()
    pltpu.sync_copy(shared_v, labels_v)
    for m in range(NUM_K):
        pltpu.sync_copy(idx_hbm.at[pl.ds(m * QB + base, _CPW)], idx_v.at[m])
    pltpu.sync_copy(zeros_hbm.at[pl.ds(base, _CPW)], out_v)
    lane = lax.iota(jnp.int32, _L)
    ones = jnp.full((_L,), 1.0, jnp.float32)
    for chunk in range(_CPW // _L):
        acc = jnp.zeros((_L,), jnp.int32)
        for m in range(NUM_K):
            iv = idx_v[m, pl.ds(chunk * _L, _L)]
            acc = acc + plsc.load_gather(labels_v, [iv])
        val = lax.shift_right_arithmetic(acc, 3)
        rows = chunk * _L + lane
        plsc.store_scatter(out_v, [rows, val], ones)
    pltpu.sync_copy(out_v, out_hbm.at[pl.ds(base, _CPW)])


def _label_onehot(labels, idx_flat, zeros):
    return pl.kernel(
        _label_onehot_body,
        out_type=jax.ShapeDtypeStruct((QB, NUM_LABELS), jnp.float32),
        mesh=plsc.VectorSubcoreMesh(core_axis_name="c", subcore_axis_name="s"),
        compiler_params=pltpu.CompilerParams(needs_layout_passes=False),
        scratch_types=[
            pltpu.VMEM_SHARED((_LPAD,), jnp.int32),
            pltpu.VMEM((_LPAD,), jnp.int32),
            pltpu.VMEM((NUM_K, _CPW), jnp.int32),
            pltpu.VMEM((_CPW, NUM_LABELS), jnp.float32),
        ],
    )(labels, idx_flat, zeros)


def kernel(x_input, train_inputs, train_labels, features):
    weighted = jnp.multiply(features, train_inputs)
    qnorm = jnp.sum(jnp.square(x_input), axis=1)[:, None]
    knorm = jnp.sum(jnp.square(weighted), axis=1)
    knorm_p = jnp.pad(knorm, (0, KPAD - knorm.shape[0]),
                      constant_values=jnp.inf)
    knorm3 = knorm_p.reshape(NSTEPS, 1, KB)
    tT = jnp.pad(train_inputs, ((0, KPAD - train_inputs.shape[0]), (0, 0))).T

    vals, idxs = _topk(x_input + x_input, qnorm, tT, knorm3)

    labels_p = jnp.pad(train_labels, (0, _LPAD - train_labels.shape[0]))
    one_hot = _label_onehot(labels_p, idxs.reshape(-1),
                            jnp.zeros((QB, NUM_LABELS), jnp.float32))
    return one_hot, vals, idxs


# speedup vs baseline: 1.1632x; 1.0190x over previous
import functools

import jax
import jax.numpy as jnp
from jax import lax
from jax.experimental import pallas as pl
from jax.experimental.pallas import tpu as pltpu
from jax.experimental.pallas import tpu_sc as plsc

NUM_K = 8
NUM_LABELS = 100
QB = 1024
KB = 2048
KPAD = 100352
NSTEPS = KPAD // KB

_NEG_INF = float("-inf")
_IMAX = 2**31 - 1


def _topk_body(x2_ref, qnorm_ref, tT_ref, knorm_ref, vals_ref, idxs_ref,
               c_ref):
    t = pl.program_id(0)

    @pl.when(t == 0)
    def _init():
        vals_ref[...] = jnp.full((QB, NUM_K), _NEG_INF, jnp.float32)
        idxs_ref[...] = jnp.full((QB, NUM_K), _IMAX, jnp.int32)

    cross2 = jnp.dot(x2_ref[...], tT_ref[...],
                     preferred_element_type=jnp.float32)
    s = cross2 - jnp.sqrt(qnorm_ref[...] + knorm_ref[0])
    col = lax.broadcasted_iota(jnp.int32, (QB, KB), 1)

    def _insert(m, im):
        v8 = vals_ref[...]
        i8 = idxs_ref[...]
        ge = (v8 > m) | ((v8 == m) & (i8 < im))
        r = jnp.sum(ge.astype(jnp.int32), axis=1, keepdims=True)
        pos = lax.broadcasted_iota(jnp.int32, (QB, NUM_K), 1)
        vsh = jnp.concatenate([m, v8[:, :NUM_K - 1]], axis=1)
        ish = jnp.concatenate([im, i8[:, :NUM_K - 1]], axis=1)
        vals_ref[...] = jnp.where(pos < r, v8, jnp.where(pos == r, m, vsh))
        idxs_ref[...] = jnp.where(pos < r, i8, jnp.where(pos == r, im, ish))

    def _extract(c, m):
        il = jnp.min(jnp.where(c == m, col, _IMAX), axis=1, keepdims=True)
        cn = jnp.where(col == il, _NEG_INF, c)
        c_ref[...] = cn
        m2 = jnp.max(cn, axis=1, keepdims=True)
        _insert(m, il + t * KB)
        go2 = jnp.any(m2 > vals_ref[:, NUM_K - 1:])
        return go2, m2

    m0 = jnp.max(s, axis=1, keepdims=True)
    carry0 = _extract(s, m0)

    def _cond(carry):
        go, _ = carry
        return go

    def _body(carry):
        _, m = carry
        return _extract(c_ref[...], m)

    lax.while_loop(_cond, _body, carry0)


def _topk(x, qnorm, tT, knorm3):
    return pl.pallas_call(
        _topk_body,
        grid=(NSTEPS,),
        in_specs=[
            pl.BlockSpec((QB, 16), lambda t: (0, 0)),
            pl.BlockSpec((QB, 1), lambda t: (0, 0)),
            pl.BlockSpec((16, KB), lambda t: (0, t)),
            pl.BlockSpec((1, 1, KB), lambda t: (t, 0, 0)),
        ],
        out_specs=[
            pl.BlockSpec((QB, NUM_K), lambda t: (0, 0)),
            pl.BlockSpec((QB, NUM_K), lambda t: (0, 0)),
        ],
        out_shape=[
            jax.ShapeDtypeStruct((QB, NUM_K), jnp.float32),
            jax.ShapeDtypeStruct((QB, NUM_K), jnp.int32),
        ],
        scratch_shapes=[pltpu.VMEM((QB, KB), jnp.float32)],
        compiler_params=pltpu.CompilerParams(
            dimension_semantics=("arbitrary",),
        ),
    )(x, qnorm, tT, knorm3)


_NC, _NS, _L = 2, 16, 16
_NW = _NC * _NS
_CPW = QB // _NW
_LPAD = 100352


def _label_onehot_body(labels_hbm, idx_hbm, zeros_hbm, out_hbm,
                       shared_v, labels_v, idx_v, out_v):
    sid = lax.axis_index("s")
    wid = sid * _NC + lax.axis_index("c")
    base = wid * _CPW

    @pl.when(sid == 0)
    def _stage():
        pltpu.sync_copy(labels_hbm, shared_v)

    plsc.subcore_barrier()
    pltpu.sync_copy(shared_v, labels_v)
    for m in range(NUM_K):
        pltpu.sync_copy(idx_hbm.at[pl.ds(m * QB + base, _CPW)], idx_v.at[m])
    pltpu.sync_copy(zeros_hbm.at[pl.ds(base, _CPW)], out_v)
    lane = lax.iota(jnp.int32, _L)
    ones = jnp.full((_L,), 1.0, jnp.float32)
    for chunk in range(_CPW // _L):
        acc = jnp.zeros((_L,), jnp.int32)
        for m in range(NUM_K):
            iv = idx_v[m, pl.ds(chunk * _L, _L)]
            acc = acc + plsc.load_gather(labels_v, [iv])
        val = lax.shift_right_arithmetic(acc, 3)
        rows = chunk * _L + lane
        plsc.store_scatter(out_v, [rows, val], ones)
    pltpu.sync_copy(out_v, out_hbm.at[pl.ds(base, _CPW)])


def _label_onehot(labels, idx_flat, zeros):
    return pl.kernel(
        _label_onehot_body,
        out_type=jax.ShapeDtypeStruct((QB, NUM_LABELS), jnp.float32),
        mesh=plsc.VectorSubcoreMesh(core_axis_name="c", subcore_axis_name="s"),
        compiler_params=pltpu.CompilerParams(needs_layout_passes=False),
        scratch_types=[
            pltpu.VMEM_SHARED((_LPAD,), jnp.int32),
            pltpu.VMEM((_LPAD,), jnp.int32),
            pltpu.VMEM((NUM_K, _CPW), jnp.int32),
            pltpu.VMEM((_CPW, NUM_LABELS), jnp.float32),
        ],
    )(labels, idx_flat, zeros)


def kernel(x_input, train_inputs, train_labels, features):
    weighted = jnp.multiply(features, train_inputs)
    qnorm = jnp.sum(jnp.square(x_input), axis=1)[:, None]
    knorm = jnp.sum(jnp.square(weighted), axis=1)
    knorm_p = jnp.pad(knorm, (0, KPAD - knorm.shape[0]),
                      constant_values=jnp.inf)
    knorm3 = knorm_p.reshape(NSTEPS, 1, KB)
    tT = jnp.pad(train_inputs, ((0, KPAD - train_inputs.shape[0]), (0, 0))).T

    vals, idxs = _topk(x_input + x_input, qnorm, tT, knorm3)

    labels_p = jnp.pad(train_labels, (0, _LPAD - train_labels.shape[0]))
    one_hot = _label_onehot(labels_p, idxs.reshape(-1),
                            jnp.zeros((QB, NUM_LABELS), jnp.float32))
    return one_hot, vals, idxs
